# baseline (speedup 1.0000x reference)
import jax
import jax.numpy as jnp
from jax import lax
from jax.experimental import pallas as pl
from jax.experimental.pallas import tpu as pltpu
from jax.experimental.pallas import tpu_sc as plsc

N = 10000
E = 320000
IN_DIM = 128
HIDDEN = 128
EMB = 64
NB = 2048

NC, NS = 2, 16
NW = NC * NS
NPAD = 10240
RPT = NPAD // NS
EPAD = 327680
CH = 128
NCHT = EPAD // CH
NCH0 = 40
NCH1 = NCHT // NS - NCH0
NBT = (NC * NB) // NW
D = HIDDEN


def _agg_body(src_hbm, dst_hbm, h_hbm, zeros_hbm, out_hbm,
              src_v, dst_v, rows_v, acc_sh, sem):
    c = lax.axis_index("c")
    s = lax.axis_index("s")
    pltpu.sync_copy(zeros_hbm, acc_sh.at[pl.ds(s * RPT, RPT)])
    @pl.when(c == 0)
    def _():
        pltpu.sync_copy(src_hbm.at[pl.ds(s * NCH0, NCH0)],
                        src_v.at[pl.ds(0, NCH0)])
        pltpu.sync_copy(dst_hbm.at[pl.ds(s * NCH0, NCH0)],
                        dst_v.at[pl.ds(0, NCH0)])

    @pl.when(c == 1)
    def _():
        pltpu.sync_copy(src_hbm.at[pl.ds(NS * NCH0 + s * NCH1, NCH1)],
                        src_v.at[pl.ds(0, NCH1)])
        pltpu.sync_copy(dst_hbm.at[pl.ds(NS * NCH0 + s * NCH1, NCH1)],
                        dst_v.at[pl.ds(0, NCH1)])

    plsc.subcore_barrier()
    nch = jnp.where(c == 0, NCH0, NCH1)

    def chunk(i, carry):
        pltpu.async_copy(h_hbm.at[src_v.at[i]], rows_v, sem).wait()
        pltpu.sync_copy(rows_v, acc_sh.at[dst_v.at[i]], add=True)
        return carry

    lax.fori_loop(0, nch, chunk, 0)
    plsc.subcore_barrier()
    pltpu.sync_copy(acc_sh.at[pl.ds(s * RPT, RPT)],
                    out_hbm.at[c, pl.ds(s * RPT, RPT)])


def _sc_agg(src3, dst3, h):
    mesh = plsc.VectorSubcoreMesh(core_axis_name="c", subcore_axis_name="s")
    zeros = jnp.zeros((RPT, D), jnp.float32)
    f = pl.kernel(
        _agg_body,
        out_type=jax.ShapeDtypeStruct((NC, NPAD, D), jnp.float32),
        mesh=mesh,
        scratch_types=[
            pltpu.VMEM((NCH1, CH), jnp.int32),
            pltpu.VMEM((NCH1, CH), jnp.int32),
            pltpu.VMEM((CH, D), jnp.float32),
            pltpu.VMEM_SHARED((NPAD, D), jnp.float32),
            pltpu.SemaphoreType.DMA,
        ],
    )
    return f(src3, dst3, h, zeros)


def _zg_body(tab_hbm, idx_hbm, out_hbm, idx_v, rows_v, sem):
    c = lax.axis_index("c")
    s = lax.axis_index("s")
    w = c * NS + s
    pltpu.sync_copy(idx_hbm.at[w], idx_v)
    pltpu.async_copy(tab_hbm.at[idx_v], rows_v, sem).wait()
    pltpu.sync_copy(rows_v, out_hbm.at[pl.ds(w * NBT, NBT)])


def _sc_zgather(tab, idx2):
    mesh = plsc.VectorSubcoreMesh(core_axis_name="c", subcore_axis_name="s")
    f = pl.kernel(
        _zg_body,
        out_type=jax.ShapeDtypeStruct((NC * NB, D), jnp.float32),
        mesh=mesh,
        scratch_types=[
            pltpu.VMEM((NBT,), jnp.int32),
            pltpu.VMEM((NBT, D), jnp.float32),
            pltpu.SemaphoreType.DMA,
        ],
    )
    return f(tab, idx2)


def _mm_body(x_ref, w_ref, o_ref):
    o_ref[...] = lax.dot_general(
        x_ref[...], w_ref[...], (((1,), (0,)), ((), ())),
        preferred_element_type=jnp.float32, precision=lax.Precision.HIGHEST)


def _mm(x, w, bm):
    m, k = x.shape
    _, n = w.shape
    return pl.pallas_call(
        _mm_body,
        grid=(m // bm,),
        in_specs=[pl.BlockSpec((bm, k), lambda i: (i, 0)),
                  pl.BlockSpec((k, n), lambda i: (0, 0))],
        out_specs=pl.BlockSpec((bm, n), lambda i: (i, 0)),
        out_shape=jax.ShapeDtypeStruct((m, n), jnp.float32),
    )(x, w)


def _add_body(p_ref, o_ref):
    o_ref[...] = p_ref[0] + p_ref[1]


def _add(p, bm):
    _, m, k = p.shape
    return pl.pallas_call(
        _add_body,
        grid=(m // bm,),
        in_specs=[pl.BlockSpec((2, bm, k), lambda i: (0, i, 0))],
        out_specs=pl.BlockSpec((bm, k), lambda i: (i, 0)),
        out_shape=jax.ShapeDtypeStruct((m, k), jnp.float32),
    )(p)


def _final_body(z_ref, w_ref, o_ref):
    zz = jnp.maximum(
        lax.dot_general(z_ref[0] + z_ref[1], w_ref[...], (((1,), (0,)), ((), ())),
                        preferred_element_type=jnp.float32,
                        precision=lax.Precision.HIGHEST),
        0.0)
    o_ref[...] = lax.dot_general(
        zz, zz, (((1,), (1,)), ((), ())),
        preferred_element_type=jnp.float32, precision=lax.Precision.HIGHEST)


def _final(z, w):
    return pl.pallas_call(
        _final_body,
        out_shape=jax.ShapeDtypeStruct((NB, NB), jnp.float32),
    )(z, w)


def kernel(adj, features, nodes_batch, W0, W1):
    pad = EPAD - E
    src = jnp.concatenate([adj[0].astype(jnp.int32),
                           jnp.zeros((pad,), jnp.int32)]).reshape(NCHT, CH)
    dst = jnp.concatenate([adj[1].astype(jnp.int32),
                           jnp.full((pad,), NPAD - 1, jnp.int32)]).reshape(NCHT, CH)
    nb = nodes_batch.astype(jnp.int32)
    idx2 = jnp.concatenate([nb, nb + NPAD]).reshape(NW, NBT)

    h1 = _mm(features, W0, 1000)
    p = _sc_agg(src, dst, h1)
    q = _add(p, 1024)
    p2 = _sc_agg(src, dst, q)
    z = _sc_zgather(p2.reshape(NC * NPAD, D), idx2).reshape(NC, NB, D)
    return _final(z, W1)


# speedup vs baseline: 1.3199x; 1.3199x over previous
import jax
import jax.numpy as jnp
from jax import lax
from jax.experimental import pallas as pl
from jax.experimental.pallas import tpu as pltpu
from jax.experimental.pallas import tpu_sc as plsc

N = 10000
E = 320000
IN_DIM = 128
HIDDEN = 128
EMB = 64
NB = 2048

NC, NS = 2, 16
NW = NC * NS
NPAD = 10240
RPT = NPAD // NS
EPAD = 327680
CH = 128
NCHT = EPAD // CH
NCH0 = 120
NCH1 = NCHT // NS - NCH0
NBT = (NC * NB) // NW
D = HIDDEN


def _agg_body(src_hbm, dst_hbm, h_hbm, zeros_hbm, out_hbm,
              src_v, dst_v, rows_v, acc_sh, sem):
    c = lax.axis_index("c")
    s = lax.axis_index("s")
    pltpu.sync_copy(zeros_hbm, acc_sh.at[pl.ds(s * RPT, RPT)])
    @pl.when(c == 0)
    def _():
        pltpu.sync_copy(src_hbm.at[pl.ds(s * NCH0, NCH0)],
                        src_v.at[pl.ds(0, NCH0)])
        pltpu.sync_copy(dst_hbm.at[pl.ds(s * NCH0, NCH0)],
                        dst_v.at[pl.ds(0, NCH0)])

    @pl.when(c == 1)
    def _():
        pltpu.sync_copy(src_hbm.at[pl.ds(NS * NCH0 + s * NCH1, NCH1)],
                        src_v.at[pl.ds(0, NCH1)])
        pltpu.sync_copy(dst_hbm.at[pl.ds(NS * NCH0 + s * NCH1, NCH1)],
                        dst_v.at[pl.ds(0, NCH1)])

    plsc.subcore_barrier()
    nch = jnp.where(c == 0, NCH0, NCH1)

    def chunk(i, carry):
        pltpu.async_copy(h_hbm.at[src_v.at[i]], rows_v, sem).wait()
        pltpu.sync_copy(rows_v, acc_sh.at[dst_v.at[i]], add=True)
        return carry

    lax.fori_loop(0, nch, chunk, 0)
    plsc.subcore_barrier()
    pltpu.sync_copy(acc_sh.at[pl.ds(s * RPT, RPT)],
                    out_hbm.at[c, pl.ds(s * RPT, RPT)])


def _sc_agg(src3, dst3, h):
    mesh = plsc.VectorSubcoreMesh(core_axis_name="c", subcore_axis_name="s")
    zeros = jnp.zeros((RPT, D), jnp.float32)
    f = pl.kernel(
        _agg_body,
        out_type=jax.ShapeDtypeStruct((NC, NPAD, D), jnp.float32),
        mesh=mesh,
        scratch_types=[
            pltpu.VMEM((max(NCH0, NCH1), CH), jnp.int32),
            pltpu.VMEM((max(NCH0, NCH1), CH), jnp.int32),
            pltpu.VMEM((CH, D), jnp.float32),
            pltpu.VMEM_SHARED((NPAD, D), jnp.float32),
            pltpu.SemaphoreType.DMA,
        ],
    )
    return f(src3, dst3, h, zeros)


def _zg_body(tab_hbm, idx_hbm, out_hbm, idx_v, rows_v, sem):
    c = lax.axis_index("c")
    s = lax.axis_index("s")
    w = c * NS + s
    pltpu.sync_copy(idx_hbm.at[w], idx_v)
    pltpu.async_copy(tab_hbm.at[idx_v], rows_v, sem).wait()
    pltpu.sync_copy(rows_v, out_hbm.at[pl.ds(w * NBT, NBT)])


def _sc_zgather(tab, idx2):
    mesh = plsc.VectorSubcoreMesh(core_axis_name="c", subcore_axis_name="s")
    f = pl.kernel(
        _zg_body,
        out_type=jax.ShapeDtypeStruct((NC * NB, D), jnp.float32),
        mesh=mesh,
        scratch_types=[
            pltpu.VMEM((NBT,), jnp.int32),
            pltpu.VMEM((NBT, D), jnp.float32),
            pltpu.SemaphoreType.DMA,
        ],
    )
    return f(tab, idx2)


def _mm_body(x_ref, w_ref, o_ref):
    o_ref[...] = lax.dot_general(
        x_ref[...], w_ref[...], (((1,), (0,)), ((), ())),
        preferred_element_type=jnp.float32, precision=lax.Precision.HIGHEST)


def _mm(x, w, bm):
    m, k = x.shape
    _, n = w.shape
    return pl.pallas_call(
        _mm_body,
        grid=(m // bm,),
        in_specs=[pl.BlockSpec((bm, k), lambda i: (i, 0)),
                  pl.BlockSpec((k, n), lambda i: (0, 0))],
        out_specs=pl.BlockSpec((bm, n), lambda i: (i, 0)),
        out_shape=jax.ShapeDtypeStruct((m, n), jnp.float32),
    )(x, w)


def _add_body(p_ref, o_ref):
    o_ref[...] = p_ref[0] + p_ref[1]


def _add(p, bm):
    _, m, k = p.shape
    return pl.pallas_call(
        _add_body,
        grid=(m // bm,),
        in_specs=[pl.BlockSpec((2, bm, k), lambda i: (0, i, 0))],
        out_specs=pl.BlockSpec((bm, k), lambda i: (i, 0)),
        out_shape=jax.ShapeDtypeStruct((m, k), jnp.float32),
    )(p)


def _final_body(z_ref, w_ref, o_ref):
    zz = jnp.maximum(
        lax.dot_general(z_ref[0] + z_ref[1], w_ref[...], (((1,), (0,)), ((), ())),
                        preferred_element_type=jnp.float32,
                        precision=lax.Precision.HIGHEST),
        0.0)
    o_ref[...] = lax.dot_general(
        zz, zz, (((1,), (1,)), ((), ())),
        preferred_element_type=jnp.float32, precision=lax.Precision.HIGHEST)


def _final(z, w):
    return pl.pallas_call(
        _final_body,
        out_shape=jax.ShapeDtypeStruct((NB, NB), jnp.float32),
    )(z, w)


def kernel(adj, features, nodes_batch, W0, W1):
    pad = EPAD - E
    src = jnp.concatenate([adj[0].astype(jnp.int32),
                           jnp.zeros((pad,), jnp.int32)]).reshape(NCHT, CH)
    dst = jnp.concatenate([adj[1].astype(jnp.int32),
                           jnp.full((pad,), NPAD - 1, jnp.int32)]).reshape(NCHT, CH)
    nb = nodes_batch.astype(jnp.int32)
    idx2 = jnp.concatenate([nb, nb + NPAD]).reshape(NW, NBT)

    h1 = _mm(features, W0, 1000)
    p = _sc_agg(src, dst, h1)
    q = _add(p, 1024)
    p2 = _sc_agg(src, dst, q)
    z = _sc_zgather(p2.reshape(NC * NPAD, D), idx2).reshape(NC, NB, D)
    return _final(z, W1)
